# Initial kernel scaffold; baseline (speedup 1.0000x reference)
import jax
import jax.numpy as jnp
from jax.experimental import pallas as pl


def kernel(queries, memory_keys, memory_contents, steps_since_last_write, accumulator, Wq, Wk, Wq_a, bq_a, Wk_a, bk_a, Wv_a, bv_a, Wo_a, bo_a):
    raise NotImplementedError("write your pallas kernel here")


# speedup vs baseline: 1.6749x; 1.6749x over previous
import functools
import math

import jax
import jax.numpy as jnp
import numpy as np
from jax.experimental import pallas as pl
from jax.experimental.pallas import tpu as pltpu

_B, _Q, _E = 2, 128, 128
_M, _C, _D = 4096, 32, 128
_S, _K, _H = 128, 8, 4
_HS = _S // _H


def _pos_enc_np():
    freqs = np.arange(0, _D, 2.0)
    inv_freq = 10000.0 ** (-freqs / _D)
    pos_seq = np.arange(_C - 1, -1, -1.0)
    sinusoid_inp = np.einsum("i,j->ij", pos_seq, inv_freq)
    return np.concatenate(
        [np.sin(sinusoid_inp), np.cos(sinusoid_inp)], axis=-1
    ).astype(np.float32)


def _select_body(q_ref, mk_ref, wq_ref, wk_ref, wqa_ref, bqa_ref,
                 idx_ref, w_ref, qa_ref):
    b = pl.program_id(0)
    q = q_ref[0]
    qh = jnp.dot(q, wq_ref[...], preferred_element_type=jnp.float32)
    kh = jnp.dot(mk_ref[0], wk_ref[...], preferred_element_type=jnp.float32)
    logits = jax.lax.dot_general(
        qh, kh, (((1,), (1,)), ((), ())),
        preferred_element_type=jnp.float32) * (1.0 / math.sqrt(_S))
    col = jax.lax.broadcasted_iota(jnp.int32, (_Q, _M), 1)
    x = logits
    vals, idxs = [], []
    neg = jnp.float32(-3.0e38)
    for _ in range(_K):
        mx = jnp.max(x, axis=1, keepdims=True)
        ismax = x >= mx
        ix = jnp.min(jnp.where(ismax, col, jnp.int32(2**30)), axis=1,
                     keepdims=True)
        vals.append(mx)
        idxs.append(ix)
        x = jnp.where(col == ix, neg, x)
    v = jnp.concatenate(vals, axis=1)
    ix = jnp.concatenate(idxs, axis=1)
    e = jnp.exp(v - v[:, :1])
    w = e / jnp.sum(e, axis=1, keepdims=True)
    idx_ref[0] = ix + b * _M
    w_ref[0] = w
    qa_ref[0] = (jnp.dot(q, wqa_ref[...], preferred_element_type=jnp.float32)
                 + bqa_ref[...])


def _select_call(queries, memory_keys, Wq, Wk, Wq_a, bq_a2):
    return pl.pallas_call(
        _select_body,
        grid=(_B,),
        in_specs=[
            pl.BlockSpec((1, _Q, _E), lambda b: (b, 0, 0)),
            pl.BlockSpec((1, _M, _D), lambda b: (b, 0, 0)),
            pl.BlockSpec((_E, _S), lambda b: (0, 0)),
            pl.BlockSpec((_D, _S), lambda b: (0, 0)),
            pl.BlockSpec((_E, _S), lambda b: (0, 0)),
            pl.BlockSpec((1, _S), lambda b: (0, 0)),
        ],
        out_specs=[
            pl.BlockSpec((1, _Q, _K), lambda b: (b, 0, 0)),
            pl.BlockSpec((1, _Q, _K), lambda b: (b, 0, 0)),
            pl.BlockSpec((1, _Q, _S), lambda b: (b, 0, 0)),
        ],
        out_shape=[
            jax.ShapeDtypeStruct((_B, _Q, _K), jnp.int32),
            jax.ShapeDtypeStruct((_B, _Q, _K), jnp.float32),
            jax.ShapeDtypeStruct((_B, _Q, _S), jnp.float32),
        ],
    )(queries, memory_keys, Wq, Wk, Wq_a, bq_a2)


def _attend_body(idx_ref, c0, c1, c2, c3, c4, c5, c6, c7, pos_ref, qa_ref,
                 w_ref, wka_ref, bka_ref, wva_ref, bva_ref, woa_ref, boa_ref,
                 out_ref):
    chunks = (c0, c1, c2, c3, c4, c5, c6, c7)
    pos = pos_ref[...]
    aug = jnp.concatenate([chunks[j][0] + pos for j in range(_K)], axis=0)
    ka = jnp.dot(aug, wka_ref[...],
                 preferred_element_type=jnp.float32) + bka_ref[...]
    va = jnp.dot(aug, wva_ref[...],
                 preferred_element_type=jnp.float32) + bva_ref[...]
    q2 = qa_ref[0]
    tmp = ka * q2
    sel = (jax.lax.broadcasted_iota(jnp.int32, (_S, _H), 0) // _HS
           == jax.lax.broadcasted_iota(jnp.int32, (_S, _H), 1)
           ).astype(jnp.float32)
    selT = (jax.lax.broadcasted_iota(jnp.int32, (_H, _S), 1) // _HS
            == jax.lax.broadcasted_iota(jnp.int32, (_H, _S), 0)
            ).astype(jnp.float32)
    seg = (jax.lax.broadcasted_iota(jnp.int32, (_K, _K * _C), 1) // _C
           == jax.lax.broadcasted_iota(jnp.int32, (_K, _K * _C), 0)
           ).astype(jnp.float32)
    logits = jnp.dot(tmp, sel,
                     preferred_element_type=jnp.float32) * (1.0 / math.sqrt(_HS))
    ws = []
    for k in range(_K):
        lk = logits[k * _C:(k + 1) * _C, :]
        m = jnp.max(lk, axis=0, keepdims=True)
        e = jnp.exp(lk - m)
        ws.append(e / jnp.sum(e, axis=0, keepdims=True))
    w_all = jnp.concatenate(ws, axis=0)
    wexp = jnp.dot(w_all, selT, preferred_element_type=jnp.float32)
    attn8 = jnp.dot(seg, wexp * va, preferred_element_type=jnp.float32)
    out8 = jnp.dot(attn8, woa_ref[...],
                   preferred_element_type=jnp.float32) + boa_ref[...]
    out_ref[0] = jnp.dot(w_ref[0], out8, preferred_element_type=jnp.float32)


def _chunk_imap(j, g, idx_ref):
    return (idx_ref[g * _K + j], 0, 0)


def _attend_call(flat_idx, mc_flat, pos, qa3, w3, Wk_a, bk2, Wv_a, bv2,
                 Wo_a, bo2):
    grid_spec = pltpu.PrefetchScalarGridSpec(
        num_scalar_prefetch=1,
        grid=(_B * _Q,),
        in_specs=[
            *[pl.BlockSpec((1, _C, _D), functools.partial(_chunk_imap, j))
              for j in range(_K)],
            pl.BlockSpec((_C, _D), lambda g, idx_ref: (0, 0)),
            pl.BlockSpec((1, 1, _S), lambda g, idx_ref: (g, 0, 0)),
            pl.BlockSpec((1, 1, _K), lambda g, idx_ref: (g, 0, 0)),
            pl.BlockSpec((_D, _S), lambda g, idx_ref: (0, 0)),
            pl.BlockSpec((1, _S), lambda g, idx_ref: (0, 0)),
            pl.BlockSpec((_D, _S), lambda g, idx_ref: (0, 0)),
            pl.BlockSpec((1, _S), lambda g, idx_ref: (0, 0)),
            pl.BlockSpec((_S, _S), lambda g, idx_ref: (0, 0)),
            pl.BlockSpec((1, _S), lambda g, idx_ref: (0, 0)),
        ],
        out_specs=pl.BlockSpec((1, 1, _S), lambda g, idx_ref: (g, 0, 0)),
    )
    return pl.pallas_call(
        _attend_body,
        grid_spec=grid_spec,
        out_shape=jax.ShapeDtypeStruct((_B * _Q, 1, _S), jnp.float32),
    )(flat_idx, mc_flat, mc_flat, mc_flat, mc_flat, mc_flat, mc_flat,
      mc_flat, mc_flat, pos, qa3, w3, Wk_a, bk2, Wv_a, bv2, Wo_a, bo2)


def kernel(queries, memory_keys, memory_contents, steps_since_last_write,
           accumulator, Wq, Wk, Wq_a, bq_a, Wk_a, bk_a, Wv_a, bv_a, Wo_a,
           bo_a):
    del steps_since_last_write, accumulator
    idx, w, qa = _select_call(queries, memory_keys, Wq, Wk, Wq_a,
                              bq_a.reshape(1, _S))
    mc_flat = memory_contents.reshape(_B * _M, _C, _D)
    flat_idx = idx.reshape(-1)
    pos = jnp.asarray(_pos_enc_np())
    qa3 = qa.reshape(_B * _Q, 1, _S)
    w3 = w.reshape(_B * _Q, 1, _K)
    out = _attend_call(flat_idx, mc_flat, pos, qa3, w3, Wk_a,
                       bk_a.reshape(1, _S), Wv_a, bv_a.reshape(1, _S),
                       Wo_a, bo_a.reshape(1, _S))
    return out.reshape(_B, _Q, _S)


# speedup vs baseline: 2.9229x; 1.7451x over previous
import functools
import math

import jax
import jax.numpy as jnp
import numpy as np
from jax.experimental import pallas as pl
from jax.experimental.pallas import tpu as pltpu

_B, _Q, _E = 2, 128, 128
_M, _C, _D = 4096, 32, 128
_S, _K, _H = 128, 8, 4
_HS = _S // _H
_G = 8
_R = _G * _K * _C
_GK = _G * _K
_GH = _G * _H
_NSTEP = (_B * _Q) // _G


def _pos_enc_np():
    freqs = np.arange(0, _D, 2.0)
    inv_freq = 10000.0 ** (-freqs / _D)
    pos_seq = np.arange(_C - 1, -1, -1.0)
    sinusoid_inp = np.einsum("i,j->ij", pos_seq, inv_freq)
    return np.concatenate(
        [np.sin(sinusoid_inp), np.cos(sinusoid_inp)], axis=-1
    ).astype(np.float32)


def _select_body(q_ref, mk_ref, wq_ref, wk_ref, wqa_ref, bqa_ref, wka_ref,
                 bka_ref, idx_ref, w_ref, u_ref, beta_ref):
    b = pl.program_id(0)
    q = q_ref[0]
    qh = jnp.dot(q, wq_ref[...], preferred_element_type=jnp.float32)
    kh = jnp.dot(mk_ref[0], wk_ref[...], preferred_element_type=jnp.float32)
    logits = jax.lax.dot_general(
        qh, kh, (((1,), (1,)), ((), ())),
        preferred_element_type=jnp.float32) * (1.0 / math.sqrt(_S))
    col = jax.lax.broadcasted_iota(jnp.int32, (_Q, _M), 1)
    x = logits
    vals, idxs = [], []
    neg = jnp.float32(-3.0e38)
    for _ in range(_K):
        mx = jnp.max(x, axis=1, keepdims=True)
        ismax = x >= mx
        ix = jnp.min(jnp.where(ismax, col, jnp.int32(2**30)), axis=1,
                     keepdims=True)
        vals.append(mx)
        idxs.append(ix)
        x = jnp.where(col == ix, neg, x)
    v = jnp.concatenate(vals, axis=1)
    ix = jnp.concatenate(idxs, axis=1)
    e = jnp.exp(v - v[:, :1])
    w = e / jnp.sum(e, axis=1, keepdims=True)
    idx_ref[0] = ix + b * _M
    w_ref[0] = w

    qa = (jnp.dot(q, wqa_ref[...], preferred_element_type=jnp.float32)
          + bqa_ref[...])
    lane = jax.lax.broadcasted_iota(jnp.int32, (_Q, _S), 1)
    for h in range(_H):
        qam = jnp.where((lane // _HS) == h, qa, 0.0)
        u_ref[h] = jax.lax.dot_general(
            qam, wka_ref[...], (((1,), (1,)), ((), ())),
            preferred_element_type=jnp.float32)
    selT = ((jax.lax.broadcasted_iota(jnp.int32, (_H, _S), 1) // _HS)
            == jax.lax.broadcasted_iota(jnp.int32, (_H, _S), 0)
            ).astype(jnp.float32)
    selTb = selT * bka_ref[...]
    beta = jax.lax.dot_general(
        selTb, qa, (((1,), (1,)), ((), ())),
        preferred_element_type=jnp.float32)
    beta_ref[:, :, 0] = beta


def _select_call(queries, memory_keys, Wq, Wk, Wq_a, bq_a2, Wk_a, bk_a2):
    return pl.pallas_call(
        _select_body,
        grid=(_B,),
        in_specs=[
            pl.BlockSpec((1, _Q, _E), lambda b: (b, 0, 0)),
            pl.BlockSpec((1, _M, _D), lambda b: (b, 0, 0)),
            pl.BlockSpec((_E, _S), lambda b: (0, 0)),
            pl.BlockSpec((_D, _S), lambda b: (0, 0)),
            pl.BlockSpec((_E, _S), lambda b: (0, 0)),
            pl.BlockSpec((1, _S), lambda b: (0, 0)),
            pl.BlockSpec((_D, _S), lambda b: (0, 0)),
            pl.BlockSpec((1, _S), lambda b: (0, 0)),
        ],
        out_specs=[
            pl.BlockSpec((1, _Q, _K), lambda b: (b, 0, 0)),
            pl.BlockSpec((1, _Q, _K), lambda b: (b, 0, 0)),
            pl.BlockSpec((_H, _Q, _D), lambda b: (0, b, 0)),
            pl.BlockSpec((_H, _Q, 1), lambda b: (0, b, 0)),
        ],
        out_shape=[
            jax.ShapeDtypeStruct((_B, _Q, _K), jnp.int32),
            jax.ShapeDtypeStruct((_B, _Q, _K), jnp.float32),
            jax.ShapeDtypeStruct((_H, _B * _Q, _D), jnp.float32),
            jax.ShapeDtypeStruct((_H, _B * _Q, 1), jnp.float32),
        ],
    )(queries, memory_keys, Wq, Wk, Wq_a, bq_a2, Wk_a, bk_a2)


def _attend_body(idx_ref, *refs):
    chunks = refs[:_GK]
    (pos_ref, u_ref, beta_ref, w3_ref, seg64_ref, seg64t_ref, exp_ref,
     wva_ref, bva_ref, woa_ref, boa_ref, out_ref) = refs[_GK:]

    pos = pos_ref[...]
    aug = jnp.concatenate([chunks[j][0] + pos for j in range(_GK)],
                          axis=0)
    u2 = u_ref[...].reshape(_GH, _D)
    lraw = jax.lax.dot_general(
        aug, u2, (((1,), (1,)), ((), ())),
        preferred_element_type=jnp.float32)

    bblk = beta_ref[:, :, 0]
    bexp = jnp.concatenate([bblk] * _H, axis=1)
    hmask = ((jax.lax.broadcasted_iota(jnp.int32, (_H, _GH), 1) // _G)
             == jax.lax.broadcasted_iota(jnp.int32, (_H, _GH), 0)
             ).astype(jnp.float32)
    betarow = jnp.dot(jnp.ones((1, _H), jnp.float32), bexp * hmask,
                      preferred_element_type=jnp.float32)

    rgrp = jax.lax.broadcasted_iota(jnp.int32, (_R, _GH), 0) // (_K * _C)
    cgrp = jax.lax.broadcasted_iota(jnp.int32, (_R, _GH), 1) % _G
    valid = rgrp == cgrp
    lm = jnp.where(valid, (lraw + betarow) * (1.0 / math.sqrt(_HS)),
                   jnp.float32(-3.0e38))
    m = jnp.max(lm)
    e = jnp.exp(lm - m)
    s = jnp.dot(seg64_ref[...], e,
                preferred_element_type=jnp.float32)
    d = jnp.dot(seg64t_ref[...], s,
                preferred_element_type=jnp.float32)
    w = e / jnp.maximum(d, jnp.float32(1e-30))
    rsum = ((jax.lax.broadcasted_iota(jnp.int32, (_GH, _H), 0) // _G)
            == jax.lax.broadcasted_iota(jnp.int32, (_GH, _H), 1)
            ).astype(jnp.float32)
    cw4 = jnp.dot(w, rsum, preferred_element_type=jnp.float32)

    w3 = w3_ref[...]
    expand = exp_ref[...]
    gmask = ((jax.lax.broadcasted_iota(jnp.int32, (_G, _R), 1) // (_K * _C))
             == jax.lax.broadcasted_iota(jnp.int32, (_G, _R), 0)
             ).astype(jnp.float32)
    segw = jnp.dot(w3, expand,
                   preferred_element_type=jnp.float32) * gmask

    zs = []
    for h in range(_H):
        yh = cw4[:, h:h + 1] * aug
        th = jnp.dot(segw, yh,
                     preferred_element_type=jnp.float32)
        zs.append(jnp.dot(th, wva_ref[:, h * _HS:(h + 1) * _HS],
                          preferred_element_type=jnp.float32))
    z = jnp.concatenate(zs, axis=1) + bva_ref[...]
    out_ref[...] = jnp.dot(z, woa_ref[...],
                           preferred_element_type=jnp.float32) + boa_ref[...]


def _chunk_imap(j, g, idx_ref):
    return (idx_ref[g * _GK + j], 0, 0)


def _attend_call(flat_idx, mc_flat, pos, u, beta, w2, seg64, seg64t, expand,
                 Wv_a, bv2, Wo_a, bo2):
    grid_spec = pltpu.PrefetchScalarGridSpec(
        num_scalar_prefetch=1,
        grid=(_NSTEP,),
        in_specs=[
            *[pl.BlockSpec((1, _C, _D), functools.partial(_chunk_imap, j))
              for j in range(_GK)],
            pl.BlockSpec((_C, _D), lambda g, idx_ref: (0, 0)),
            pl.BlockSpec((_H, _G, _D), lambda g, idx_ref: (0, g, 0)),
            pl.BlockSpec((_H, _G, 1), lambda g, idx_ref: (0, g, 0)),
            pl.BlockSpec((_G, _K), lambda g, idx_ref: (g, 0)),
            pl.BlockSpec((_GK, _R), lambda g, idx_ref: (0, 0)),
            pl.BlockSpec((_R, _GK), lambda g, idx_ref: (0, 0)),
            pl.BlockSpec((_K, _R), lambda g, idx_ref: (0, 0)),
            pl.BlockSpec((_D, _S), lambda g, idx_ref: (0, 0)),
            pl.BlockSpec((1, _S), lambda g, idx_ref: (0, 0)),
            pl.BlockSpec((_S, _S), lambda g, idx_ref: (0, 0)),
            pl.BlockSpec((1, _S), lambda g, idx_ref: (0, 0)),
        ],
        out_specs=pl.BlockSpec((_G, _S), lambda g, idx_ref: (g, 0)),
    )
    return pl.pallas_call(
        _attend_body,
        grid_spec=grid_spec,
        out_shape=jax.ShapeDtypeStruct((_B * _Q, _S), jnp.float32),
    )(flat_idx, *([mc_flat] * _GK), pos, u, beta, w2, seg64, seg64t, expand,
      Wv_a, bv2, Wo_a, bo2)


def _np_consts():
    r = np.arange(_R)
    seg64 = (r[None, :] // _C == np.arange(_GK)[:, None]).astype(np.float32)
    seg64t = seg64.T.copy()
    expand = ((r[None, :] // _C) % _K
              == np.arange(_K)[:, None]).astype(np.float32)
    return seg64, seg64t, expand


def kernel(queries, memory_keys, memory_contents, steps_since_last_write,
           accumulator, Wq, Wk, Wq_a, bq_a, Wk_a, bk_a, Wv_a, bv_a, Wo_a,
           bo_a):
    del steps_since_last_write, accumulator
    idx, w, u, beta = _select_call(
        queries, memory_keys, Wq, Wk, Wq_a, bq_a.reshape(1, _S), Wk_a,
        bk_a.reshape(1, _S))
    mc_flat = memory_contents.reshape(_B * _M, _C, _D)
    flat_idx = idx.reshape(-1)
    pos = jnp.asarray(_pos_enc_np())
    seg64, seg64t, expand = _np_consts()
    out = _attend_call(flat_idx, mc_flat, pos, u, beta,
                       w.reshape(_B * _Q, _K), jnp.asarray(seg64),
                       jnp.asarray(seg64t), jnp.asarray(expand), Wv_a,
                       bv_a.reshape(1, _S), Wo_a, bo_a.reshape(1, _S))
    return out.reshape(_B, _Q, _S)


# speedup vs baseline: 3.5973x; 1.2307x over previous
import functools
import math

import jax
import jax.numpy as jnp
import numpy as np
from jax.experimental import pallas as pl
from jax.experimental.pallas import tpu as pltpu

_B, _Q, _E = 2, 128, 128
_M, _C, _D = 4096, 32, 128
_S, _K, _H = 128, 8, 4
_HS = _S // _H
_G = 8
_R = _G * _K * _C
_GK = _G * _K
_GH = _G * _H
_NSTEP = (_B * _Q) // _G


def _pos_enc_np():
    freqs = np.arange(0, _D, 2.0)
    inv_freq = 10000.0 ** (-freqs / _D)
    pos_seq = np.arange(_C - 1, -1, -1.0)
    sinusoid_inp = np.einsum("i,j->ij", pos_seq, inv_freq)
    return np.concatenate(
        [np.sin(sinusoid_inp), np.cos(sinusoid_inp)], axis=-1
    ).astype(np.float32)


def _select_body(q_ref, mk_ref, wq_ref, wk_ref, wqa_ref, bqa_ref, wka_ref,
                 bka_ref, idx_ref, w_ref, u_ref, beta_ref):
    b = pl.program_id(0)
    q = q_ref[0]
    qh = jnp.dot(q, wq_ref[...], preferred_element_type=jnp.float32)
    kh = jnp.dot(mk_ref[0], wk_ref[...], preferred_element_type=jnp.float32)
    logits = jax.lax.dot_general(
        qh, kh, (((1,), (1,)), ((), ())),
        preferred_element_type=jnp.float32) * (1.0 / math.sqrt(_S))
    col = jax.lax.broadcasted_iota(jnp.int32, (_Q, _M), 1)
    x = logits
    vals, idxs = [], []
    neg = jnp.float32(-3.0e38)
    for _ in range(_K):
        mx = jnp.max(x, axis=1, keepdims=True)
        ismax = x >= mx
        ix = jnp.min(jnp.where(ismax, col, jnp.int32(2**30)), axis=1,
                     keepdims=True)
        vals.append(mx)
        idxs.append(ix)
        x = jnp.where(col == ix, neg, x)
    v = jnp.concatenate(vals, axis=1)
    ix = jnp.concatenate(idxs, axis=1)
    e = jnp.exp(v - v[:, :1])
    w = e / jnp.sum(e, axis=1, keepdims=True)
    idx_ref[0] = ix + b * _M
    w_ref[0] = w

    qa = (jnp.dot(q, wqa_ref[...], preferred_element_type=jnp.float32)
          + bqa_ref[...])
    lane = jax.lax.broadcasted_iota(jnp.int32, (_Q, _S), 1)
    for h in range(_H):
        qam = jnp.where((lane // _HS) == h, qa, 0.0)
        u_ref[h] = jax.lax.dot_general(
            qam, wka_ref[...], (((1,), (1,)), ((), ())),
            preferred_element_type=jnp.float32)
    selT = ((jax.lax.broadcasted_iota(jnp.int32, (_H, _S), 1) // _HS)
            == jax.lax.broadcasted_iota(jnp.int32, (_H, _S), 0)
            ).astype(jnp.float32)
    selTb = selT * bka_ref[...]
    beta = jax.lax.dot_general(
        selTb, qa, (((1,), (1,)), ((), ())),
        preferred_element_type=jnp.float32)
    beta_ref[:, :, 0] = beta


def _select_call(queries, memory_keys, Wq, Wk, Wq_a, bq_a2, Wk_a, bk_a2):
    return pl.pallas_call(
        _select_body,
        grid=(_B,),
        in_specs=[
            pl.BlockSpec((1, _Q, _E), lambda b: (b, 0, 0)),
            pl.BlockSpec((1, _M, _D), lambda b: (b, 0, 0)),
            pl.BlockSpec((_E, _S), lambda b: (0, 0)),
            pl.BlockSpec((_D, _S), lambda b: (0, 0)),
            pl.BlockSpec((_E, _S), lambda b: (0, 0)),
            pl.BlockSpec((1, _S), lambda b: (0, 0)),
            pl.BlockSpec((_D, _S), lambda b: (0, 0)),
            pl.BlockSpec((1, _S), lambda b: (0, 0)),
        ],
        out_specs=[
            pl.BlockSpec((1, _Q, _K), lambda b: (b, 0, 0)),
            pl.BlockSpec((1, _Q, _K), lambda b: (b, 0, 0)),
            pl.BlockSpec((_H, _Q, _D), lambda b: (0, b, 0)),
            pl.BlockSpec((_H, _Q, 1), lambda b: (0, b, 0)),
        ],
        out_shape=[
            jax.ShapeDtypeStruct((_B, _Q, _K), jnp.int32),
            jax.ShapeDtypeStruct((_B, _Q, _K), jnp.float32),
            jax.ShapeDtypeStruct((_H, _B * _Q, _D), jnp.float32),
            jax.ShapeDtypeStruct((_H, _B * _Q, 1), jnp.float32),
        ],
    )(queries, memory_keys, Wq, Wk, Wq_a, bq_a2, Wk_a, bk_a2)


def _attend_body(idx_ref, *refs):
    chunks = refs[:_GK]
    (pos_ref, u_ref, beta_ref, w3_ref, seg64_ref, seg64t_ref,
     wva_ref, bva_ref, woa_ref, boa_ref, out_ref) = refs[_GK:]

    pos = pos_ref[...]
    aug = jnp.concatenate([chunks[j][0] + pos for j in range(_GK)],
                          axis=0)
    u2 = u_ref[...].reshape(_GH, _D)
    lraw = jax.lax.dot_general(
        aug, u2, (((1,), (1,)), ((), ())),
        preferred_element_type=jnp.float32)

    bblk = beta_ref[:, :, 0]
    bexp = jnp.concatenate([bblk] * _H, axis=1)
    hmask = ((jax.lax.broadcasted_iota(jnp.int32, (_H, _GH), 1) // _G)
             == jax.lax.broadcasted_iota(jnp.int32, (_H, _GH), 0)
             ).astype(jnp.float32)
    betarow = jnp.dot(jnp.ones((1, _H), jnp.float32), bexp * hmask,
                      preferred_element_type=jnp.float32)

    rgrp = jax.lax.broadcasted_iota(jnp.int32, (_R, _GH), 0) // (_K * _C)
    cgrp = jax.lax.broadcasted_iota(jnp.int32, (_R, _GH), 1) % _G
    valid = rgrp == cgrp
    lm = jnp.where(valid, (lraw + betarow) * (1.0 / math.sqrt(_HS)),
                   jnp.float32(-3.0e38))
    m = jnp.max(lm)
    e = jnp.exp(lm - m)

    w3 = w3_ref[...]
    gsel = ((jax.lax.broadcasted_iota(jnp.int32, (_GK, _G), 0) // _K)
            == jax.lax.broadcasted_iota(jnp.int32, (_GK, _G), 1)
            ).astype(jnp.float32)
    kmask = ((jax.lax.broadcasted_iota(jnp.int32, (_GK, _K), 0) % _K)
             == jax.lax.broadcasted_iota(jnp.int32, (_GK, _K), 1)
             ).astype(jnp.float32)
    w3rep = jnp.dot(gsel, w3, preferred_element_type=jnp.float32) * kmask
    wfull = jnp.dot(w3rep, jnp.ones((_K, _GH), jnp.float32),
                    preferred_element_type=jnp.float32)

    s = jnp.dot(seg64_ref[...], e,
                preferred_element_type=jnp.float32)
    s2 = s / jnp.maximum(wfull, jnp.float32(1e-30))
    d2 = jnp.dot(seg64t_ref[...], s2,
                 preferred_element_type=jnp.float32)
    cw = e / jnp.maximum(d2, jnp.float32(1e-30))

    t3 = jax.lax.dot_general(
        aug, cw, (((0,), (0,)), ((), ())),
        preferred_element_type=jnp.float32)
    zs = []
    for h in range(_H):
        zs.append(jax.lax.dot_general(
            t3[:, h * _G:(h + 1) * _G], wva_ref[:, h * _HS:(h + 1) * _HS],
            (((0,), (0,)), ((), ())),
            preferred_element_type=jnp.float32))
    z = jnp.concatenate(zs, axis=1) + bva_ref[...]
    out_ref[...] = jnp.dot(z, woa_ref[...],
                           preferred_element_type=jnp.float32) + boa_ref[...]


def _chunk_imap(j, g, idx_ref):
    return (idx_ref[g * _GK + j], 0, 0)


def _attend_call(flat_idx, mc_flat, pos, u, beta, w2, seg64, seg64t,
                 Wv_a, bv2, Wo_a, bo2):
    grid_spec = pltpu.PrefetchScalarGridSpec(
        num_scalar_prefetch=1,
        grid=(_NSTEP,),
        in_specs=[
            *[pl.BlockSpec((1, _C, _D), functools.partial(_chunk_imap, j))
              for j in range(_GK)],
            pl.BlockSpec((_C, _D), lambda g, idx_ref: (0, 0)),
            pl.BlockSpec((_H, _G, _D), lambda g, idx_ref: (0, g, 0)),
            pl.BlockSpec((_H, _G, 1), lambda g, idx_ref: (0, g, 0)),
            pl.BlockSpec((_G, _K), lambda g, idx_ref: (g, 0)),
            pl.BlockSpec((_GK, _R), lambda g, idx_ref: (0, 0)),
            pl.BlockSpec((_R, _GK), lambda g, idx_ref: (0, 0)),
            pl.BlockSpec((_D, _S), lambda g, idx_ref: (0, 0)),
            pl.BlockSpec((1, _S), lambda g, idx_ref: (0, 0)),
            pl.BlockSpec((_S, _S), lambda g, idx_ref: (0, 0)),
            pl.BlockSpec((1, _S), lambda g, idx_ref: (0, 0)),
        ],
        out_specs=pl.BlockSpec((_G, _S), lambda g, idx_ref: (g, 0)),
    )
    return pl.pallas_call(
        _attend_body,
        grid_spec=grid_spec,
        out_shape=jax.ShapeDtypeStruct((_B * _Q, _S), jnp.float32),
    )(flat_idx, *([mc_flat] * _GK), pos, u, beta, w2, seg64, seg64t,
      Wv_a, bv2, Wo_a, bo2)


def _np_consts():
    r = np.arange(_R)
    seg64 = (r[None, :] // _C == np.arange(_GK)[:, None]).astype(np.float32)
    seg64t = seg64.T.copy()
    return seg64, seg64t


def kernel(queries, memory_keys, memory_contents, steps_since_last_write,
           accumulator, Wq, Wk, Wq_a, bq_a, Wk_a, bk_a, Wv_a, bv_a, Wo_a,
           bo_a):
    del steps_since_last_write, accumulator
    idx, w, u, beta = _select_call(
        queries, memory_keys, Wq, Wk, Wq_a, bq_a.reshape(1, _S), Wk_a,
        bk_a.reshape(1, _S))
    mc_flat = memory_contents.reshape(_B * _M, _C, _D)
    flat_idx = idx.reshape(-1)
    pos = jnp.asarray(_pos_enc_np())
    seg64, seg64t = _np_consts()
    out = _attend_call(flat_idx, mc_flat, pos, u, beta,
                       w.reshape(_B * _Q, _K), jnp.asarray(seg64),
                       jnp.asarray(seg64t), Wv_a,
                       bv_a.reshape(1, _S), Wo_a, bo_a.reshape(1, _S))
    return out.reshape(_B, _Q, _S)


# speedup vs baseline: 4.4455x; 1.2358x over previous
import functools
import math

import jax
import jax.numpy as jnp
import numpy as np
from jax.experimental import pallas as pl
from jax.experimental.pallas import tpu as pltpu

_B, _Q, _E = 2, 128, 128
_M, _C, _D = 4096, 32, 128
_S, _K, _H = 128, 8, 4
_HS = _S // _H
_G = 8
_R = _G * _K * _C
_GK = _G * _K
_GH = _G * _H
_NSTEP = (_B * _Q) // _G


def _pos_enc_np():
    freqs = np.arange(0, _D, 2.0)
    inv_freq = 10000.0 ** (-freqs / _D)
    pos_seq = np.arange(_C - 1, -1, -1.0)
    sinusoid_inp = np.einsum("i,j->ij", pos_seq, inv_freq)
    return np.concatenate(
        [np.sin(sinusoid_inp), np.cos(sinusoid_inp)], axis=-1
    ).astype(np.float32)


def _select_body(q_ref, mk_ref, wq_ref, wk_ref, wqa_ref, bqa_ref, wka_ref,
                 bka_ref, idx_ref, w_ref, u_ref, beta_ref):
    b = pl.program_id(0)
    q = q_ref[0]
    qh = jnp.dot(q, wq_ref[...], preferred_element_type=jnp.float32)
    kh = jnp.dot(mk_ref[0], wk_ref[...], preferred_element_type=jnp.float32)
    logits = jax.lax.dot_general(
        qh, kh, (((1,), (1,)), ((), ())),
        preferred_element_type=jnp.float32) * (1.0 / math.sqrt(_S))
    col = jax.lax.broadcasted_iota(jnp.int32, (_Q, _M), 1)
    x = logits
    vals, idxs = [], []
    neg = jnp.float32(-3.0e38)
    for _ in range(_K):
        mx = jnp.max(x, axis=1, keepdims=True)
        ismax = x >= mx
        ix = jnp.min(jnp.where(ismax, col, jnp.int32(2**30)), axis=1,
                     keepdims=True)
        vals.append(mx)
        idxs.append(ix)
        x = jnp.where(col == ix, neg, x)
    v = jnp.concatenate(vals, axis=1)
    ix = jnp.concatenate(idxs, axis=1)
    e = jnp.exp(v - v[:, :1])
    w = e / jnp.sum(e, axis=1, keepdims=True)
    idx_ref[0] = ix + b * _M
    w_ref[0] = w

    qa = (jnp.dot(q, wqa_ref[...], preferred_element_type=jnp.float32)
          + bqa_ref[...])
    lane = jax.lax.broadcasted_iota(jnp.int32, (_Q, _S), 1)
    for h in range(_H):
        qam = jnp.where((lane // _HS) == h, qa, 0.0)
        u_ref[h] = jax.lax.dot_general(
            qam, wka_ref[...], (((1,), (1,)), ((), ())),
            preferred_element_type=jnp.float32)
    sel = ((jax.lax.broadcasted_iota(jnp.int32, (_S, _H), 0) // _HS)
           == jax.lax.broadcasted_iota(jnp.int32, (_S, _H), 1)
           ).astype(jnp.float32)
    beta_ref[...] = jnp.dot(qa * bka_ref[...], sel,
                            preferred_element_type=jnp.float32)


def _select_call(queries, memory_keys, Wq, Wk, Wq_a, bq_a2, Wk_a, bk_a2):
    return pl.pallas_call(
        _select_body,
        grid=(_B,),
        in_specs=[
            pl.BlockSpec((1, _Q, _E), lambda b: (b, 0, 0)),
            pl.BlockSpec((1, _M, _D), lambda b: (b, 0, 0)),
            pl.BlockSpec((_E, _S), lambda b: (0, 0)),
            pl.BlockSpec((_D, _S), lambda b: (0, 0)),
            pl.BlockSpec((_E, _S), lambda b: (0, 0)),
            pl.BlockSpec((1, _S), lambda b: (0, 0)),
            pl.BlockSpec((_D, _S), lambda b: (0, 0)),
            pl.BlockSpec((1, _S), lambda b: (0, 0)),
        ],
        out_specs=[
            pl.BlockSpec((1, _Q, _K), lambda b: (b, 0, 0)),
            pl.BlockSpec((1, _Q, _K), lambda b: (b, 0, 0)),
            pl.BlockSpec((_H, _Q, _D), lambda b: (0, b, 0)),
            pl.BlockSpec((_Q, _H), lambda b: (b, 0)),
        ],
        out_shape=[
            jax.ShapeDtypeStruct((_B, _Q, _K), jnp.int32),
            jax.ShapeDtypeStruct((_B, _Q, _K), jnp.float32),
            jax.ShapeDtypeStruct((_H, _B * _Q, _D), jnp.float32),
            jax.ShapeDtypeStruct((_B * _Q, _H), jnp.float32),
        ],
    )(queries, memory_keys, Wq, Wk, Wq_a, bq_a2, Wk_a, bk_a2)


def _attend_body(idx_ref, *refs):
    chunks = refs[:_GK]
    (pos_ref, u_ref, beta_ref, w3_ref, seg64_ref, seg64t_ref, vld_ref,
     wva_ref, bva_ref, woa_ref, boa_ref, out_ref) = refs[_GK:]

    pos = pos_ref[...]
    aug = jnp.concatenate([chunks[j][0] + pos for j in range(_GK)],
                          axis=0)
    u2 = u_ref[...].reshape(_GH, _D)
    lraw = jax.lax.dot_general(
        u2, aug, (((1,), (1,)), ((), ())),
        preferred_element_type=jnp.float32)

    bqh = beta_ref[...]
    asel = ((jax.lax.broadcasted_iota(jnp.int32, (_GH, _G), 0) % _G)
            == jax.lax.broadcasted_iota(jnp.int32, (_GH, _G), 1)
            ).astype(jnp.float32)
    hmask = ((jax.lax.broadcasted_iota(jnp.int32, (_GH, _H), 0) // _G)
             == jax.lax.broadcasted_iota(jnp.int32, (_GH, _H), 1)
             ).astype(jnp.float32)
    bcol = jnp.dot(jnp.dot(asel, bqh,
                           preferred_element_type=jnp.float32) * hmask,
                   jnp.ones((_H, 1), jnp.float32),
                   preferred_element_type=jnp.float32)

    lsc = (lraw + bcol) * (1.0 / math.sqrt(_HS))
    mt = jnp.max(lsc, axis=1, keepdims=True)
    e = jnp.exp(lsc - mt) * vld_ref[...]

    w3 = w3_ref[...]
    w3e = jnp.concatenate([w3] * _G, axis=1)
    pmask = ((jax.lax.broadcasted_iota(jnp.int32, (_G, _GK), 1) // _K)
             == jax.lax.broadcasted_iota(jnp.int32, (_G, _GK), 0)
             ).astype(jnp.float32)
    wrow = jnp.dot(jnp.ones((1, _G), jnp.float32), w3e * pmask,
                   preferred_element_type=jnp.float32)

    s = jax.lax.dot_general(
        e, seg64t_ref[...], (((1,), (0,)), ((), ())),
        preferred_element_type=jnp.float32)
    s2 = s / jnp.maximum(wrow, jnp.float32(1e-30))
    d2 = jnp.dot(s2, seg64_ref[...],
                 preferred_element_type=jnp.float32)
    cw = e / jnp.maximum(d2, jnp.float32(1e-30))

    t3 = jnp.dot(cw, aug,
                 preferred_element_type=jnp.float32)
    zs = []
    for h in range(_H):
        zs.append(jnp.dot(
            t3[h * _G:(h + 1) * _G, :], wva_ref[:, h * _HS:(h + 1) * _HS],
            preferred_element_type=jnp.float32))
    z = jnp.concatenate(zs, axis=1) + bva_ref[...]
    out_ref[...] = jnp.dot(z, woa_ref[...],
                           preferred_element_type=jnp.float32) + boa_ref[...]


def _chunk_imap(j, g, idx_ref):
    return (idx_ref[g * _GK + j], 0, 0)


def _attend_call(flat_idx, mc_flat, pos, u, beta, w2, seg64, seg64t, vld,
                 Wv_a, bv2, Wo_a, bo2):
    grid_spec = pltpu.PrefetchScalarGridSpec(
        num_scalar_prefetch=1,
        grid=(_NSTEP,),
        in_specs=[
            *[pl.BlockSpec((1, _C, _D), functools.partial(_chunk_imap, j))
              for j in range(_GK)],
            pl.BlockSpec((_C, _D), lambda g, idx_ref: (0, 0)),
            pl.BlockSpec((_H, _G, _D), lambda g, idx_ref: (0, g, 0)),
            pl.BlockSpec((_G, _H), lambda g, idx_ref: (g, 0)),
            pl.BlockSpec((_G, _K), lambda g, idx_ref: (g, 0)),
            pl.BlockSpec((_GK, _R), lambda g, idx_ref: (0, 0)),
            pl.BlockSpec((_R, _GK), lambda g, idx_ref: (0, 0)),
            pl.BlockSpec((_GH, _R), lambda g, idx_ref: (0, 0)),
            pl.BlockSpec((_D, _S), lambda g, idx_ref: (0, 0)),
            pl.BlockSpec((1, _S), lambda g, idx_ref: (0, 0)),
            pl.BlockSpec((_S, _S), lambda g, idx_ref: (0, 0)),
            pl.BlockSpec((1, _S), lambda g, idx_ref: (0, 0)),
        ],
        out_specs=pl.BlockSpec((_G, _S), lambda g, idx_ref: (g, 0)),
    )
    return pl.pallas_call(
        _attend_body,
        grid_spec=grid_spec,
        out_shape=jax.ShapeDtypeStruct((_B * _Q, _S), jnp.float32),
    )(flat_idx, *([mc_flat] * _GK), pos, u, beta, w2, seg64, seg64t, vld,
      Wv_a, bv2, Wo_a, bo2)


def _np_consts():
    r = np.arange(_R)
    seg64 = (r[None, :] // _C == np.arange(_GK)[:, None]).astype(np.float32)
    seg64t = seg64.T.copy()
    j = np.arange(_GH)
    vld = ((j[:, None] % _G) == (r[None, :] // (_K * _C))).astype(np.float32)
    return seg64, seg64t, vld


def kernel(queries, memory_keys, memory_contents, steps_since_last_write,
           accumulator, Wq, Wk, Wq_a, bq_a, Wk_a, bk_a, Wv_a, bv_a, Wo_a,
           bo_a):
    del steps_since_last_write, accumulator
    idx, w, u, beta = _select_call(
        queries, memory_keys, Wq, Wk, Wq_a, bq_a.reshape(1, _S), Wk_a,
        bk_a.reshape(1, _S))
    mc_flat = memory_contents.reshape(_B * _M, _C, _D)
    flat_idx = idx.reshape(-1)
    pos = jnp.asarray(_pos_enc_np())
    seg64, seg64t, vld = _np_consts()
    out = _attend_call(flat_idx, mc_flat, pos, u, beta,
                       w.reshape(_B * _Q, _K), jnp.asarray(seg64),
                       jnp.asarray(seg64t), jnp.asarray(vld), Wv_a,
                       bv_a.reshape(1, _S), Wo_a, bo_a.reshape(1, _S))
    return out.reshape(_B, _Q, _S)


# speedup vs baseline: 5.5392x; 1.2460x over previous
import functools
import math

import jax
import jax.numpy as jnp
import numpy as np
from jax.experimental import pallas as pl
from jax.experimental.pallas import tpu as pltpu

_B, _Q, _E = 2, 128, 128
_M, _C, _D = 4096, 32, 128
_S, _K, _H = 128, 8, 4
_HS = _S // _H
_G = 8
_R = _G * _K * _C
_GK = _G * _K
_GH = _G * _H
_NSTEP = (_B * _Q) // _G


def _pos_enc_np():
    freqs = np.arange(0, _D, 2.0)
    inv_freq = 10000.0 ** (-freqs / _D)
    pos_seq = np.arange(_C - 1, -1, -1.0)
    sinusoid_inp = np.einsum("i,j->ij", pos_seq, inv_freq)
    return np.concatenate(
        [np.sin(sinusoid_inp), np.cos(sinusoid_inp)], axis=-1
    ).astype(np.float32)


def _select_body(q_ref, mk_ref, wq_ref, wk_ref, wqa_ref, bqa_ref, wka_ref,
                 bka_ref, idx_ref, w_ref, u_ref, beta_ref):
    b = pl.program_id(0)
    q = q_ref[0]
    qh = jnp.dot(q, wq_ref[...], preferred_element_type=jnp.float32)
    kh = jnp.dot(mk_ref[0], wk_ref[...], preferred_element_type=jnp.float32)
    logits = jax.lax.dot_general(
        qh, kh, (((1,), (1,)), ((), ())),
        preferred_element_type=jnp.float32) * (1.0 / math.sqrt(_S))
    col = jax.lax.broadcasted_iota(jnp.int32, (_Q, _M), 1)
    x = logits
    vals, idxs = [], []
    neg = jnp.float32(-3.0e38)
    for _ in range(_K):
        mx = jnp.max(x, axis=1, keepdims=True)
        ismax = x >= mx
        ix = jnp.min(jnp.where(ismax, col, jnp.int32(2**30)), axis=1,
                     keepdims=True)
        vals.append(mx)
        idxs.append(ix)
        x = jnp.where(col == ix, neg, x)
    v = jnp.concatenate(vals, axis=1)
    ix = jnp.concatenate(idxs, axis=1)
    e = jnp.exp(v - v[:, :1])
    w = e / jnp.sum(e, axis=1, keepdims=True)
    idx_ref[0] = ix + b * _M
    w_ref[0] = w

    qa = (jnp.dot(q, wqa_ref[...], preferred_element_type=jnp.float32)
          + bqa_ref[...])
    lane = jax.lax.broadcasted_iota(jnp.int32, (_Q, _S), 1)
    for h in range(_H):
        qam = jnp.where((lane // _HS) == h, qa, 0.0)
        u_ref[h] = jax.lax.dot_general(
            qam, wka_ref[...], (((1,), (1,)), ((), ())),
            preferred_element_type=jnp.float32)
    sel = ((jax.lax.broadcasted_iota(jnp.int32, (_S, _H), 0) // _HS)
           == jax.lax.broadcasted_iota(jnp.int32, (_S, _H), 1)
           ).astype(jnp.float32)
    beta_ref[...] = jnp.dot(qa * bka_ref[...], sel,
                            preferred_element_type=jnp.float32)


def _select_call(queries, memory_keys, Wq, Wk, Wq_a, bq_a2, Wk_a, bk_a2):
    return pl.pallas_call(
        _select_body,
        grid=(_B,),
        in_specs=[
            pl.BlockSpec((1, _Q, _E), lambda b: (b, 0, 0)),
            pl.BlockSpec((1, _M, _D), lambda b: (b, 0, 0)),
            pl.BlockSpec((_E, _S), lambda b: (0, 0)),
            pl.BlockSpec((_D, _S), lambda b: (0, 0)),
            pl.BlockSpec((_E, _S), lambda b: (0, 0)),
            pl.BlockSpec((1, _S), lambda b: (0, 0)),
            pl.BlockSpec((_D, _S), lambda b: (0, 0)),
            pl.BlockSpec((1, _S), lambda b: (0, 0)),
        ],
        out_specs=[
            pl.BlockSpec((1, _Q, _K), lambda b: (b, 0, 0)),
            pl.BlockSpec((1, _Q, _K), lambda b: (b, 0, 0)),
            pl.BlockSpec((_H, _Q, _D), lambda b: (0, b, 0)),
            pl.BlockSpec((_Q, _H), lambda b: (b, 0)),
        ],
        out_shape=[
            jax.ShapeDtypeStruct((_B, _Q, _K), jnp.int32),
            jax.ShapeDtypeStruct((_B, _Q, _K), jnp.float32),
            jax.ShapeDtypeStruct((_H, _B * _Q, _D), jnp.float32),
            jax.ShapeDtypeStruct((_B * _Q, _H), jnp.float32),
        ],
    )(queries, memory_keys, Wq, Wk, Wq_a, bq_a2, Wk_a, bk_a2)


def _attend_body(idx_ref, mc_ref, pos_ref, u_ref, beta_ref, w3_ref,
                 seg64_ref, seg64t_ref, vld_ref, wva_ref, bva_ref, woa_ref,
                 boa_ref, out_ref, abuf, sem):
    i = pl.program_id(0)
    slot = jax.lax.rem(i, 2)
    nslot = jax.lax.rem(i + 1, 2)

    def _issue(step, slot_):
        for j in range(_GK):
            ix = idx_ref[step * _GK + j]
            pltpu.make_async_copy(
                mc_ref.at[ix], abuf.at[slot_, j], sem.at[slot_]).start()

    @pl.when(i == 0)
    def _():
        _issue(i, slot)

    @pl.when(i + 1 < _NSTEP)
    def _():
        _issue(i + 1, nslot)

    pltpu.make_async_copy(
        mc_ref.at[pl.ds(0, _GK)], abuf.at[slot], sem.at[slot]).wait()

    chunks = abuf[slot].reshape(_R, _D)
    pos = pos_ref[...]
    u2 = u_ref[...].reshape(_GH, _D)
    ctile = ((jax.lax.broadcasted_iota(jnp.int32, (_C, _R), 1) % _C)
             == jax.lax.broadcasted_iota(jnp.int32, (_C, _R), 0)
             ).astype(jnp.float32)
    pu = jax.lax.dot_general(
        u2, pos, (((1,), (1,)), ((), ())),
        preferred_element_type=jnp.float32)
    lraw = jax.lax.dot_general(
        u2, chunks, (((1,), (1,)), ((), ())),
        preferred_element_type=jnp.float32) + jnp.dot(
            pu, ctile, preferred_element_type=jnp.float32)

    bqh = beta_ref[...]
    asel = ((jax.lax.broadcasted_iota(jnp.int32, (_GH, _G), 0) % _G)
            == jax.lax.broadcasted_iota(jnp.int32, (_GH, _G), 1)
            ).astype(jnp.float32)
    hmask = ((jax.lax.broadcasted_iota(jnp.int32, (_GH, _H), 0) // _G)
             == jax.lax.broadcasted_iota(jnp.int32, (_GH, _H), 1)
             ).astype(jnp.float32)
    bcol = jnp.dot(jnp.dot(asel, bqh,
                           preferred_element_type=jnp.float32) * hmask,
                   jnp.ones((_H, 1), jnp.float32),
                   preferred_element_type=jnp.float32)

    lsc = (lraw + bcol) * (1.0 / math.sqrt(_HS))
    mt = jnp.max(lsc, axis=1, keepdims=True)
    e = jnp.exp(lsc - mt) * vld_ref[...]

    w3 = w3_ref[...]
    w3e = jnp.concatenate([w3] * _G, axis=1)
    pmask = ((jax.lax.broadcasted_iota(jnp.int32, (_G, _GK), 1) // _K)
             == jax.lax.broadcasted_iota(jnp.int32, (_G, _GK), 0)
             ).astype(jnp.float32)
    wrow = jnp.dot(jnp.ones((1, _G), jnp.float32), w3e * pmask,
                   preferred_element_type=jnp.float32)

    s = jax.lax.dot_general(
        e, seg64t_ref[...], (((1,), (0,)), ((), ())),
        preferred_element_type=jnp.float32)
    s2 = s / jnp.maximum(wrow, jnp.float32(1e-30))
    d2 = jnp.dot(s2, seg64_ref[...],
                 preferred_element_type=jnp.float32)
    cw = e / jnp.maximum(d2, jnp.float32(1e-30))

    csum = jax.lax.dot_general(
        cw, ctile, (((1,), (1,)), ((), ())),
        preferred_element_type=jnp.float32)
    t3 = jnp.dot(cw, chunks,
                 preferred_element_type=jnp.float32) + jnp.dot(
        csum, pos, preferred_element_type=jnp.float32)
    zs = []
    for h in range(_H):
        zs.append(jnp.dot(
            t3[h * _G:(h + 1) * _G, :], wva_ref[:, h * _HS:(h + 1) * _HS],
            preferred_element_type=jnp.float32))
    z = jnp.concatenate(zs, axis=1) + bva_ref[...]
    out_ref[...] = jnp.dot(z, woa_ref[...],
                           preferred_element_type=jnp.float32) + boa_ref[...]


def _attend_call(flat_idx, mc_flat, pos, u, beta, w2, seg64, seg64t, vld,
                 Wv_a, bv2, Wo_a, bo2):
    grid_spec = pltpu.PrefetchScalarGridSpec(
        num_scalar_prefetch=1,
        grid=(_NSTEP,),
        in_specs=[
            pl.BlockSpec(memory_space=pl.ANY),
            pl.BlockSpec((_C, _D), lambda g, idx_ref: (0, 0)),
            pl.BlockSpec((_H, _G, _D), lambda g, idx_ref: (0, g, 0)),
            pl.BlockSpec((_G, _H), lambda g, idx_ref: (g, 0)),
            pl.BlockSpec((_G, _K), lambda g, idx_ref: (g, 0)),
            pl.BlockSpec((_GK, _R), lambda g, idx_ref: (0, 0)),
            pl.BlockSpec((_R, _GK), lambda g, idx_ref: (0, 0)),
            pl.BlockSpec((_GH, _R), lambda g, idx_ref: (0, 0)),
            pl.BlockSpec((_D, _S), lambda g, idx_ref: (0, 0)),
            pl.BlockSpec((1, _S), lambda g, idx_ref: (0, 0)),
            pl.BlockSpec((_S, _S), lambda g, idx_ref: (0, 0)),
            pl.BlockSpec((1, _S), lambda g, idx_ref: (0, 0)),
        ],
        out_specs=pl.BlockSpec((_G, _S), lambda g, idx_ref: (g, 0)),
        scratch_shapes=[
            pltpu.VMEM((2, _GK, _C, _D), jnp.float32),
            pltpu.SemaphoreType.DMA((2,)),
        ],
    )
    return pl.pallas_call(
        _attend_body,
        grid_spec=grid_spec,
        out_shape=jax.ShapeDtypeStruct((_B * _Q, _S), jnp.float32),
    )(flat_idx, mc_flat, pos, u, beta, w2, seg64, seg64t, vld,
      Wv_a, bv2, Wo_a, bo2)


def _np_consts():
    r = np.arange(_R)
    seg64 = (r[None, :] // _C == np.arange(_GK)[:, None]).astype(np.float32)
    seg64t = seg64.T.copy()
    j = np.arange(_GH)
    vld = ((j[:, None] % _G) == (r[None, :] // (_K * _C))).astype(np.float32)
    return seg64, seg64t, vld


def kernel(queries, memory_keys, memory_contents, steps_since_last_write,
           accumulator, Wq, Wk, Wq_a, bq_a, Wk_a, bk_a, Wv_a, bv_a, Wo_a,
           bo_a):
    del steps_since_last_write, accumulator
    idx, w, u, beta = _select_call(
        queries, memory_keys, Wq, Wk, Wq_a, bq_a.reshape(1, _S), Wk_a,
        bk_a.reshape(1, _S))
    mc_flat = memory_contents.reshape(_B * _M, _C, _D)
    flat_idx = idx.reshape(-1)
    pos = jnp.asarray(_pos_enc_np())
    seg64, seg64t, vld = _np_consts()
    out = _attend_call(flat_idx, mc_flat, pos, u, beta,
                       w.reshape(_B * _Q, _K), jnp.asarray(seg64),
                       jnp.asarray(seg64t), jnp.asarray(vld), Wv_a,
                       bv_a.reshape(1, _S), Wo_a, bo_a.reshape(1, _S))
    return out.reshape(_B, _Q, _S)
